# baseline (speedup 1.0000x reference)
import jax
import jax.numpy as jnp
import numpy as np
from jax.experimental import pallas as pl
from jax.experimental.pallas import tpu as pltpu
from jax.experimental.pallas import tpu_sc as plsc

_B = 16
_N = 4096
_S = 512
_K = 16
_R2 = np.float32(0.1 * 0.1)
_F32 = jnp.float32
_TW = 128


_BH = _B // 2


def _fps_body(x_ref, y_ref, z_ref, ox_ref, oy_ref, oz_ref, dmin_ref):
    iota_n = jax.lax.broadcasted_iota(jnp.int32, (_BH, _N), 1)
    iota_s = jax.lax.broadcasted_iota(jnp.int32, (_BH, _S), 1)
    ox_ref[...] = jnp.zeros((_BH, _S), _F32)
    oy_ref[...] = jnp.zeros((_BH, _S), _F32)
    oz_ref[...] = jnp.zeros((_BH, _S), _F32)
    dmin_ref[...] = jnp.full((_BH, _N), 1e10, _F32)

    def step(s, carry):
        cx, cy, cz = carry
        e = (iota_s == s).astype(_F32)
        ox_ref[...] = ox_ref[...] + cx * e
        oy_ref[...] = oy_ref[...] + cy * e
        oz_ref[...] = oz_ref[...] + cz * e
        X = x_ref[...]
        Y = y_ref[...]
        Z = z_ref[...]
        d = ((X - cx) ** 2 + (Y - cy) ** 2) + (Z - cz) ** 2
        dm = jnp.minimum(dmin_ref[...], d)
        dmin_ref[...] = dm
        m = jnp.max(dm, axis=1, keepdims=True)
        cand = jnp.where(dm == m, iota_n, _N)
        fi = jnp.min(cand, axis=1, keepdims=True)
        oh = iota_n == fi
        nx = jnp.sum(jnp.where(oh, X, 0.0), axis=1, keepdims=True)
        ny = jnp.sum(jnp.where(oh, Y, 0.0), axis=1, keepdims=True)
        nz = jnp.sum(jnp.where(oh, Z, 0.0), axis=1, keepdims=True)
        return nx, ny, nz

    c0 = (x_ref[:, 0:1], y_ref[:, 0:1], z_ref[:, 0:1])
    jax.lax.fori_loop(0, _S, step, c0)


def _fps(X, Y, Z):
    out = [jax.ShapeDtypeStruct((_B, _S), _F32)] * 3
    bspec_in = pl.BlockSpec((_BH, _N), lambda i: (i, 0))
    bspec_out = pl.BlockSpec((_BH, _S), lambda i: (i, 0))
    return pl.pallas_call(
        _fps_body,
        grid=(2,),
        in_specs=[bspec_in] * 3,
        out_specs=[bspec_out] * 3,
        out_shape=out,
        scratch_shapes=[pltpu.VMEM((_BH, _N), _F32)],
        compiler_params=pltpu.CompilerParams(
            dimension_semantics=("parallel",)),
    )(X, Y, Z)


def _bq_body(q_ref, p_ref, o_ref):
    b = pl.program_id(0)
    Q = q_ref[0]
    P = p_ref[0]
    D = -2.0 * jnp.dot(Q, P, preferred_element_type=_F32)
    qx, qy, qz = Q[:, 0:1], Q[:, 1:2], Q[:, 2:3]
    sq = (qx * qx + qy * qy) + qz * qz
    X = P[0:1, :]
    Y = P[1:2, :]
    Z = P[2:3, :]
    sx = X * X + Y * Y + Z * Z
    D = (D + sq) + sx
    iota_n = jax.lax.broadcasted_iota(jnp.int32, (_S, _N), 1)
    val = jnp.where(D <= _R2, iota_n, _N)
    col16 = jax.lax.broadcasted_iota(jnp.int32, (_S, _K), 1)
    acc = jnp.full((_S, _K), _N, jnp.int32)
    for k in range(_K):
        m = jnp.min(val, axis=1, keepdims=True)
        acc = jnp.where(col16 == k, m, acc)
        val = jnp.where(val == m, _N, val)
    first = acc[:, 0:1]
    acc = jnp.where(acc == _N, first, acc)
    acc = jnp.where(acc == _N, _N - 1, acc)
    o_ref[0] = acc + b * _N


def _ball_query(q8, xyz8):
    return pl.pallas_call(
        _bq_body,
        grid=(_B,),
        in_specs=[
            pl.BlockSpec((1, _S, 8), lambda b: (b, 0, 0)),
            pl.BlockSpec((1, 8, _N), lambda b: (b, 0, 0)),
        ],
        out_specs=pl.BlockSpec((1, _S, _K), lambda b: (b, 0, 0)),
        out_shape=jax.ShapeDtypeStruct((_B, _S, _K), jnp.int32),
        compiler_params=pltpu.CompilerParams(
            dimension_semantics=("parallel",)),
    )(q8, xyz8)


def _sc_gather(table, indices):
    num = indices.shape[0]
    idx2 = indices.reshape(1, num)
    mesh = plsc.VectorSubcoreMesh(core_axis_name="core",
                                  subcore_axis_name="subcore")

    @pl.kernel(out_type=jax.ShapeDtypeStruct((num, table.shape[1]),
                                             table.dtype),
               mesh=mesh)
    def _k(x_hbm, i_hbm, o_hbm):
        def body(i_vmem, o_vmem):
            pltpu.sync_copy(x_hbm.at[i_vmem.at[0]], o_vmem)

        pltpu.emit_pipeline(
            body,
            grid=(num // 128,),
            in_specs=[pl.BlockSpec((1, 128), index_map=lambda i: (0, i))],
            out_specs=[pl.BlockSpec((128, table.shape[1]),
                                    index_map=lambda i: (i, 0))],
            core_axis_name=("core", "subcore"),
            dimension_semantics=(pltpu.PARALLEL,),
        )(i_hbm, o_hbm)

    return _k(table, idx2)


_RB = 4096
_NROW = _B * _S * _K
_NBLK = _NROW // _RB


def _p1_body(g_ref, q_ref, w_ref, pb_ref, l_ref, s_ref, acc_ref):
    i = pl.program_id(0)

    @pl.when(i == 0)
    def _():
        acc_ref[...] = jnp.zeros_like(acc_ref)

    G16 = g_ref[:, :16].reshape(_RB // _K, _K, 16)
    H = (G16 - q_ref[...].reshape(_RB // _K, 1, 16)).reshape(_RB, 16)
    L = jnp.dot(H, w_ref[...], preferred_element_type=_F32) + pb_ref[0:1, :]
    l_ref[...] = L
    acc_ref[0:1, :] = acc_ref[0:1, :] + jnp.sum(L, axis=0, keepdims=True)
    acc_ref[1:2, :] = acc_ref[1:2, :] + jnp.sum(L * L, axis=0, keepdims=True)

    @pl.when(i == _NBLK - 1)
    def _():
        s_ref[...] = acc_ref[...]


def _p1(G, Qfull, W0T, pb0):
    c = W0T.shape[1]
    return pl.pallas_call(
        _p1_body,
        grid=(_NBLK,),
        in_specs=[
            pl.BlockSpec((_RB, _TW), lambda i: (i, 0)),
            pl.BlockSpec((_RB // _K, 16), lambda i: (i, 0)),
            pl.BlockSpec((16, c), lambda i: (0, 0)),
            pl.BlockSpec((8, c), lambda i: (0, 0)),
        ],
        out_specs=[
            pl.BlockSpec((_RB, c), lambda i: (i, 0)),
            pl.BlockSpec((8, c), lambda i: (0, 0)),
        ],
        out_shape=[
            jax.ShapeDtypeStruct((_NROW, c), _F32),
            jax.ShapeDtypeStruct((8, c), _F32),
        ],
        scratch_shapes=[pltpu.VMEM((8, c), _F32)],
    )(G, Qfull, W0T, pb0)


def _bn_relu(L, stats, pb):
    n = _F32(_NROW)
    mean = stats[0:1, :] / n
    var = stats[1:2, :] / n - mean * mean
    return jax.nn.relu(
        pb[1:2, :] * (L - mean) / jnp.sqrt(var + 1e-5) + pb[2:3, :])


def _p2_body(l_ref, s_ref, pb_ref, w_ref, pbn_ref, ln_ref, sn_ref, acc_ref):
    i = pl.program_id(0)

    @pl.when(i == 0)
    def _():
        acc_ref[...] = jnp.zeros_like(acc_ref)

    H = _bn_relu(l_ref[...], s_ref[...], pb_ref[...])
    L = jnp.dot(H, w_ref[...], preferred_element_type=_F32) + pbn_ref[0:1, :]
    ln_ref[...] = L
    acc_ref[0:1, :] = acc_ref[0:1, :] + jnp.sum(L, axis=0, keepdims=True)
    acc_ref[1:2, :] = acc_ref[1:2, :] + jnp.sum(L * L, axis=0, keepdims=True)

    @pl.when(i == _NBLK - 1)
    def _():
        sn_ref[...] = acc_ref[...]


def _p2(L, stats, pb, WT, pbn):
    c = WT.shape[0]
    cn = WT.shape[1]
    return pl.pallas_call(
        _p2_body,
        grid=(_NBLK,),
        in_specs=[
            pl.BlockSpec((_RB, c), lambda i: (i, 0)),
            pl.BlockSpec((8, c), lambda i: (0, 0)),
            pl.BlockSpec((8, c), lambda i: (0, 0)),
            pl.BlockSpec((c, cn), lambda i: (0, 0)),
            pl.BlockSpec((8, cn), lambda i: (0, 0)),
        ],
        out_specs=[
            pl.BlockSpec((_RB, cn), lambda i: (i, 0)),
            pl.BlockSpec((8, cn), lambda i: (0, 0)),
        ],
        out_shape=[
            jax.ShapeDtypeStruct((_NROW, cn), _F32),
            jax.ShapeDtypeStruct((8, cn), _F32),
        ],
        scratch_shapes=[pltpu.VMEM((8, cn), _F32)],
    )(L, stats, pb, WT, pbn)


def _p4_body(l_ref, s_ref, pb_ref, o_ref):
    H = _bn_relu(l_ref[...], s_ref[...], pb_ref[...])
    o_ref[...] = jnp.max(H.reshape(_RB // _K, _K, 32), axis=1)


def _p4(L2, stats, pb):
    return pl.pallas_call(
        _p4_body,
        grid=(_NBLK,),
        in_specs=[
            pl.BlockSpec((_RB, 32), lambda i: (i, 0)),
            pl.BlockSpec((8, 32), lambda i: (0, 0)),
            pl.BlockSpec((8, 32), lambda i: (0, 0)),
        ],
        out_specs=pl.BlockSpec((_RB // _K, 32), lambda i: (i, 0)),
        out_shape=jax.ShapeDtypeStruct((_B * _S, 32), _F32),
    )(L2, stats, pb)


def _pbrow(b, g, be, c):
    p = jnp.zeros((8, c), _F32)
    p = p.at[0, : b.shape[0]].set(b)
    p = p.at[1, : g.shape[0]].set(g)
    p = p.at[2, : be.shape[0]].set(be)
    return p


def kernel(pts, W0, b0, gamma0, beta0, W1, b1, gamma1, beta1,
           W2, b2, gamma2, beta2):
    X = pts[:, 0, :]
    Y = pts[:, 1, :]
    Z = pts[:, 2, :]

    ox, oy, oz = _fps(X, Y, Z)

    q3 = jnp.stack([ox, oy, oz], axis=-1)
    q8 = jnp.pad(q3, ((0, 0), (0, 0), (0, 5)))
    xyz8 = jnp.pad(jnp.stack([X, Y, Z], axis=1),
                   ((0, 0), (0, 5), (0, 0)))
    gidx = _ball_query(q8, xyz8)

    table = jnp.pad(jnp.transpose(pts, (0, 2, 1)).reshape(_B * _N, 6),
                    ((0, 0), (0, _TW - 6)))
    G = _sc_gather(table, gidx.reshape(_NROW))

    qrows = jnp.pad(q3.reshape(_B * _S, 3), ((0, 0), (0, 13)))

    W0T = jnp.pad(W0, ((0, 0), (0, 10))).T
    W1T = W1.T
    W2T = W2.T
    pb0 = _pbrow(b0, gamma0, beta0, 16)
    pb1 = _pbrow(b1, gamma1, beta1, 16)
    pb2 = _pbrow(b2, gamma2, beta2, 32)

    L0, s0 = _p1(G, qrows, W0T, pb0)
    L1, s1 = _p2(L0, s0, pb0, W1T, pb1)
    L2, s2 = _p2(L1, s1, pb1, W2T, pb2)
    M = _p4(L2, s2, pb2)

    new_xyz = jnp.stack([ox, oy, oz], axis=1)
    new_points = jnp.transpose(M.reshape(_B, _S, 32), (0, 2, 1))
    return new_xyz, new_points


# speedup vs baseline: 1.2295x; 1.2295x over previous
import jax
import jax.numpy as jnp
import numpy as np
from jax.experimental import pallas as pl
from jax.experimental.pallas import tpu as pltpu
from jax.experimental.pallas import tpu_sc as plsc

_B = 16
_N = 4096
_S = 512
_K = 16
_R2 = np.float32(0.1 * 0.1)
_F32 = jnp.float32
_TW = 128


def _fps_body(x_ref, y_ref, z_ref, ox_ref, oy_ref, oz_ref, dmin_ref):
    iota_n = jax.lax.broadcasted_iota(jnp.int32, (_B, _N), 1)
    iota_s = jax.lax.broadcasted_iota(jnp.int32, (_B, _S), 1)
    ox_ref[...] = jnp.zeros((_B, _S), _F32)
    oy_ref[...] = jnp.zeros((_B, _S), _F32)
    oz_ref[...] = jnp.zeros((_B, _S), _F32)
    dmin_ref[...] = jnp.full((_B, _N), 1e10, _F32)

    def step(s, carry):
        cx, cy, cz = carry
        e = (iota_s == s).astype(_F32)
        ox_ref[...] = ox_ref[...] + cx * e
        oy_ref[...] = oy_ref[...] + cy * e
        oz_ref[...] = oz_ref[...] + cz * e
        X = x_ref[...]
        Y = y_ref[...]
        Z = z_ref[...]
        d = ((X - cx) ** 2 + (Y - cy) ** 2) + (Z - cz) ** 2
        dm = jnp.minimum(dmin_ref[...], d)
        dmin_ref[...] = dm
        m = jnp.max(dm, axis=1, keepdims=True)
        cand = jnp.where(dm == m, iota_n, _N)
        fi = jnp.min(cand, axis=1, keepdims=True)
        oh = iota_n == fi
        nx = jnp.sum(jnp.where(oh, X, 0.0), axis=1, keepdims=True)
        ny = jnp.sum(jnp.where(oh, Y, 0.0), axis=1, keepdims=True)
        nz = jnp.sum(jnp.where(oh, Z, 0.0), axis=1, keepdims=True)
        return nx, ny, nz

    c0 = (x_ref[:, 0:1], y_ref[:, 0:1], z_ref[:, 0:1])
    jax.lax.fori_loop(0, _S, step, c0)


def _fps(X, Y, Z):
    out = [jax.ShapeDtypeStruct((_B, _S), _F32)] * 3
    return pl.pallas_call(
        _fps_body,
        out_shape=out,
        scratch_shapes=[pltpu.VMEM((_B, _N), _F32)],
    )(X, Y, Z)


def _bq_body(q_ref, p_ref, o_ref):
    b = pl.program_id(0)
    Q = q_ref[0]
    P = p_ref[0]
    D = -2.0 * jnp.dot(Q, P, preferred_element_type=_F32)
    qx, qy, qz = Q[:, 0:1], Q[:, 1:2], Q[:, 2:3]
    sq = (qx * qx + qy * qy) + qz * qz
    X = P[0:1, :]
    Y = P[1:2, :]
    Z = P[2:3, :]
    sx = X * X + Y * Y + Z * Z
    D = (D + sq) + sx
    iota_n = jax.lax.broadcasted_iota(jnp.int32, (_S, _N), 1)
    val = jnp.where(D <= _R2, iota_n, _N)
    col16 = jax.lax.broadcasted_iota(jnp.int32, (_S, _K), 1)
    acc = jnp.full((_S, _K), _N, jnp.int32)
    for k in range(_K):
        m = jnp.min(val, axis=1, keepdims=True)
        acc = jnp.where(col16 == k, m, acc)
        val = jnp.where(val == m, _N, val)
    first = acc[:, 0:1]
    acc = jnp.where(acc == _N, first, acc)
    acc = jnp.where(acc == _N, _N - 1, acc)
    o_ref[0] = acc + b * _N


def _ball_query(q8, xyz8):
    return pl.pallas_call(
        _bq_body,
        grid=(_B,),
        in_specs=[
            pl.BlockSpec((1, _S, 8), lambda b: (b, 0, 0)),
            pl.BlockSpec((1, 8, _N), lambda b: (b, 0, 0)),
        ],
        out_specs=pl.BlockSpec((1, _S, _K), lambda b: (b, 0, 0)),
        out_shape=jax.ShapeDtypeStruct((_B, _S, _K), jnp.int32),
    )(q8, xyz8)


def _sc_gather(table, indices):
    num = indices.shape[0]
    idx2 = indices.reshape(1, num)
    mesh = plsc.VectorSubcoreMesh(core_axis_name="core",
                                  subcore_axis_name="subcore")

    @pl.kernel(out_type=jax.ShapeDtypeStruct((num, table.shape[1]),
                                             table.dtype),
               mesh=mesh)
    def _k(x_hbm, i_hbm, o_hbm):
        def body(i_vmem, o_vmem):
            pltpu.sync_copy(x_hbm.at[i_vmem.at[0]], o_vmem)

        pltpu.emit_pipeline(
            body,
            grid=(num // 128,),
            in_specs=[pl.BlockSpec((1, 128), index_map=lambda i: (0, i))],
            out_specs=[pl.BlockSpec((128, table.shape[1]),
                                    index_map=lambda i: (i, 0))],
            core_axis_name=("core", "subcore"),
            dimension_semantics=(pltpu.PARALLEL,),
        )(i_hbm, o_hbm)

    return _k(table, idx2)


_RB = 4096
_NROW = _B * _S * _K
_NBLK = _NROW // _RB


def _p1_body(g_ref, q_ref, w_ref, pb_ref, l_ref, s_ref, acc_ref):
    i = pl.program_id(0)

    @pl.when(i == 0)
    def _():
        acc_ref[...] = jnp.zeros_like(acc_ref)

    G16 = g_ref[:, :16].reshape(_RB // _K, _K, 16)
    H = (G16 - q_ref[...].reshape(_RB // _K, 1, 16)).reshape(_RB, 16)
    L = jnp.dot(H, w_ref[...], preferred_element_type=_F32) + pb_ref[0:1, :]
    l_ref[...] = L
    acc_ref[0:1, :] = acc_ref[0:1, :] + jnp.sum(L, axis=0, keepdims=True)
    acc_ref[1:2, :] = acc_ref[1:2, :] + jnp.sum(L * L, axis=0, keepdims=True)

    @pl.when(i == _NBLK - 1)
    def _():
        s_ref[...] = acc_ref[...]


def _p1(G, Qfull, W0T, pb0):
    c = W0T.shape[1]
    return pl.pallas_call(
        _p1_body,
        grid=(_NBLK,),
        in_specs=[
            pl.BlockSpec((_RB, _TW), lambda i: (i, 0)),
            pl.BlockSpec((_RB // _K, 16), lambda i: (i, 0)),
            pl.BlockSpec((16, c), lambda i: (0, 0)),
            pl.BlockSpec((8, c), lambda i: (0, 0)),
        ],
        out_specs=[
            pl.BlockSpec((_RB, c), lambda i: (i, 0)),
            pl.BlockSpec((8, c), lambda i: (0, 0)),
        ],
        out_shape=[
            jax.ShapeDtypeStruct((_NROW, c), _F32),
            jax.ShapeDtypeStruct((8, c), _F32),
        ],
        scratch_shapes=[pltpu.VMEM((8, c), _F32)],
    )(G, Qfull, W0T, pb0)


def _bn_relu(L, stats, pb):
    n = _F32(_NROW)
    mean = stats[0:1, :] / n
    var = stats[1:2, :] / n - mean * mean
    return jax.nn.relu(
        pb[1:2, :] * (L - mean) / jnp.sqrt(var + 1e-5) + pb[2:3, :])


def _p2_body(l_ref, s_ref, pb_ref, w_ref, pbn_ref, ln_ref, sn_ref, acc_ref):
    i = pl.program_id(0)

    @pl.when(i == 0)
    def _():
        acc_ref[...] = jnp.zeros_like(acc_ref)

    H = _bn_relu(l_ref[...], s_ref[...], pb_ref[...])
    L = jnp.dot(H, w_ref[...], preferred_element_type=_F32) + pbn_ref[0:1, :]
    ln_ref[...] = L
    acc_ref[0:1, :] = acc_ref[0:1, :] + jnp.sum(L, axis=0, keepdims=True)
    acc_ref[1:2, :] = acc_ref[1:2, :] + jnp.sum(L * L, axis=0, keepdims=True)

    @pl.when(i == _NBLK - 1)
    def _():
        sn_ref[...] = acc_ref[...]


def _p2(L, stats, pb, WT, pbn):
    c = WT.shape[0]
    cn = WT.shape[1]
    return pl.pallas_call(
        _p2_body,
        grid=(_NBLK,),
        in_specs=[
            pl.BlockSpec((_RB, c), lambda i: (i, 0)),
            pl.BlockSpec((8, c), lambda i: (0, 0)),
            pl.BlockSpec((8, c), lambda i: (0, 0)),
            pl.BlockSpec((c, cn), lambda i: (0, 0)),
            pl.BlockSpec((8, cn), lambda i: (0, 0)),
        ],
        out_specs=[
            pl.BlockSpec((_RB, cn), lambda i: (i, 0)),
            pl.BlockSpec((8, cn), lambda i: (0, 0)),
        ],
        out_shape=[
            jax.ShapeDtypeStruct((_NROW, cn), _F32),
            jax.ShapeDtypeStruct((8, cn), _F32),
        ],
        scratch_shapes=[pltpu.VMEM((8, cn), _F32)],
    )(L, stats, pb, WT, pbn)


def _p4_body(l_ref, s_ref, pb_ref, o_ref):
    H = _bn_relu(l_ref[...], s_ref[...], pb_ref[...])
    o_ref[...] = jnp.max(H.reshape(_RB // _K, _K, 32), axis=1)


def _p4(L2, stats, pb):
    return pl.pallas_call(
        _p4_body,
        grid=(_NBLK,),
        in_specs=[
            pl.BlockSpec((_RB, 32), lambda i: (i, 0)),
            pl.BlockSpec((8, 32), lambda i: (0, 0)),
            pl.BlockSpec((8, 32), lambda i: (0, 0)),
        ],
        out_specs=pl.BlockSpec((_RB // _K, 32), lambda i: (i, 0)),
        out_shape=jax.ShapeDtypeStruct((_B * _S, 32), _F32),
    )(L2, stats, pb)


def _pbrow(b, g, be, c):
    p = jnp.zeros((8, c), _F32)
    p = p.at[0, : b.shape[0]].set(b)
    p = p.at[1, : g.shape[0]].set(g)
    p = p.at[2, : be.shape[0]].set(be)
    return p


def kernel(pts, W0, b0, gamma0, beta0, W1, b1, gamma1, beta1,
           W2, b2, gamma2, beta2):
    X = pts[:, 0, :]
    Y = pts[:, 1, :]
    Z = pts[:, 2, :]

    ox, oy, oz = _fps(X, Y, Z)

    q3 = jnp.stack([ox, oy, oz], axis=-1)
    q8 = jnp.pad(q3, ((0, 0), (0, 0), (0, 5)))
    xyz8 = jnp.pad(jnp.stack([X, Y, Z], axis=1),
                   ((0, 0), (0, 5), (0, 0)))
    gidx = _ball_query(q8, xyz8)

    table = jnp.pad(jnp.transpose(pts, (0, 2, 1)).reshape(_B * _N, 6),
                    ((0, 0), (0, _TW - 6)))
    G = _sc_gather(table, gidx.reshape(_NROW))

    qrows = jnp.pad(q3.reshape(_B * _S, 3), ((0, 0), (0, 13)))

    W0T = jnp.pad(W0, ((0, 0), (0, 10))).T
    W1T = W1.T
    W2T = W2.T
    pb0 = _pbrow(b0, gamma0, beta0, 16)
    pb1 = _pbrow(b1, gamma1, beta1, 16)
    pb2 = _pbrow(b2, gamma2, beta2, 32)

    L0, s0 = _p1(G, qrows, W0T, pb0)
    L1, s1 = _p2(L0, s0, pb0, W1T, pb1)
    L2, s2 = _p2(L1, s1, pb1, W2T, pb2)
    M = _p4(L2, s2, pb2)

    new_xyz = jnp.stack([ox, oy, oz], axis=1)
    new_points = jnp.transpose(M.reshape(_B, _S, 32), (0, 2, 1))
    return new_xyz, new_points
